# Initial kernel scaffold; baseline (speedup 1.0000x reference)
import jax
import jax.numpy as jnp
from jax.experimental import pallas as pl


def kernel(x, table_lr, table_deep, gamma, beta, W1, b1, W2, b2, W3, b3):
    raise NotImplementedError("write your pallas kernel here")


# speedup vs baseline: 2.3714x; 2.3714x over previous
import functools

import jax
import jax.numpy as jnp
from jax import lax
from jax.experimental import pallas as pl
from jax.experimental.pallas import tpu as pltpu
from jax.experimental.pallas import tpu_sc as plsc

_B = 4096
_F = 26
_E = 16
_D = _F * _E
_BF = _B * _F

_NC = 2
_NS = 16
_NW = _NC * _NS
_CHUNK = 128
_NCHUNK = _BF // (_NW * _CHUNK)


def _sc_gather_body(idx_hbm, tlr_hbm, tdp_hbm, wide_hbm, deep_hbm,
                    idx_v, rows_lr, rows_dp, sem_lr, sem_dp):
    wid = lax.axis_index("s") * _NC + lax.axis_index("c")
    pltpu.sync_copy(idx_hbm.at[wid], idx_v)

    def issue(j, carry):
        pltpu.async_copy(tlr_hbm.at[idx_v.at[j]], rows_lr.at[j], sem_lr)
        pltpu.async_copy(tdp_hbm.at[idx_v.at[j]], rows_dp.at[j], sem_dp)
        return carry

    lax.fori_loop(0, _NCHUNK, issue, 0)
    pltpu.make_async_copy(wide_hbm.at[wid], rows_lr, sem_lr).wait()
    pltpu.sync_copy(rows_lr, wide_hbm.at[wid])
    pltpu.make_async_copy(deep_hbm.at[wid], rows_dp, sem_dp).wait()
    pltpu.sync_copy(rows_dp, deep_hbm.at[wid])


@functools.cache
def _make_sc_gather():
    return pl.kernel(
        _sc_gather_body,
        out_type=[
            jax.ShapeDtypeStruct((_NW, _NCHUNK, _CHUNK, _E), jnp.float32),
            jax.ShapeDtypeStruct((_NW, _NCHUNK, _CHUNK, _E), jnp.float32),
        ],
        mesh=plsc.VectorSubcoreMesh(core_axis_name="c", subcore_axis_name="s"),
        compiler_params=pltpu.CompilerParams(
            use_tc_tiling_on_sc=False, needs_layout_passes=False),
        scratch_types=[
            pltpu.VMEM((_NCHUNK, _CHUNK), jnp.int32),
            pltpu.VMEM((_NCHUNK, _CHUNK, _E), jnp.float32),
            pltpu.VMEM((_NCHUNK, _CHUNK, _E), jnp.float32),
            pltpu.SemaphoreType.DMA,
            pltpu.SemaphoreType.DMA,
        ],
    )


_BLK = 512
_NBLK = _B // _BLK


def _stats_body(deep_ref, acc_ref):
    i = pl.program_id(0)
    blk = deep_ref[...]
    s = jnp.sum(blk, axis=0, keepdims=True)
    q = jnp.sum(blk * blk, axis=0, keepdims=True)
    sq = jnp.concatenate([s, q], axis=0)

    @pl.when(i == 0)
    def _():
        acc_ref[...] = sq

    @pl.when(i != 0)
    def _():
        acc_ref[...] += sq


def _mlp_body(stats_ref, gamma_ref, beta_ref, deep_ref, wide_ref,
              w1_ref, b1_ref, w2_ref, b2_ref, w3_ref, b3_ref, out_ref):
    inv_b = 1.0 / _B
    mean = stats_ref[0:1, :] * inv_b
    var = stats_ref[1:2, :] * inv_b - mean * mean
    scale = gamma_ref[...] * lax.rsqrt(var + 1e-5)
    shift = beta_ref[...] - mean * scale
    h = deep_ref[...] * scale + shift
    h1 = jnp.maximum(
        jnp.dot(h, w1_ref[...], preferred_element_type=jnp.float32)
        + b1_ref[...], 0.0)
    h2 = jnp.maximum(
        jnp.dot(h1, w2_ref[...], preferred_element_type=jnp.float32)
        + b2_ref[...], 0.0)
    d = jnp.sum(h2 * w3_ref[...], axis=1, keepdims=True) + b3_ref[...]
    out_ref[...] = jax.nn.sigmoid(wide_ref[...] + d)


def _tc_stats(deep):
    return pl.pallas_call(
        _stats_body,
        grid=(_NBLK,),
        in_specs=[pl.BlockSpec((_BLK, _D), lambda i: (i, 0))],
        out_specs=pl.BlockSpec((2, _D), lambda i: (0, 0)),
        out_shape=jax.ShapeDtypeStruct((2, _D), jnp.float32),
    )(deep)


def _tc_mlp(stats, gamma, beta, deep, wide, w1, b1, w2, b2, w3, b3):
    fixed = lambda i: (0, 0)
    return pl.pallas_call(
        _mlp_body,
        grid=(_NBLK,),
        in_specs=[
            pl.BlockSpec((2, _D), fixed),
            pl.BlockSpec((1, _D), fixed),
            pl.BlockSpec((1, _D), fixed),
            pl.BlockSpec((_BLK, _D), lambda i: (i, 0)),
            pl.BlockSpec((_BLK, _D), lambda i: (i, 0)),
            pl.BlockSpec((_D, 1024), fixed),
            pl.BlockSpec((1, 1024), fixed),
            pl.BlockSpec((1024, 512), fixed),
            pl.BlockSpec((1, 512), fixed),
            pl.BlockSpec((1, 512), fixed),
            pl.BlockSpec((1, 1), fixed),
        ],
        out_specs=pl.BlockSpec((_BLK, _D), lambda i: (i, 0)),
        out_shape=jax.ShapeDtypeStruct((_B, _D), jnp.float32),
    )(stats, gamma, beta, deep, wide, w1, b1, w2, b2, w3, b3)


def kernel(x, table_lr, table_deep, gamma, beta, W1, b1, W2, b2, W3, b3):
    idx = x.astype(jnp.int32).reshape(_NW, _NCHUNK, _CHUNK)
    wide3, deep3 = _make_sc_gather()(idx, table_lr, table_deep)
    wide = wide3.reshape(_B, _D)
    deep = deep3.reshape(_B, _D)
    stats = _tc_stats(deep)
    return _tc_mlp(stats, gamma.reshape(1, _D), beta.reshape(1, _D),
                   deep, wide, W1, b1.reshape(1, 1024), W2,
                   b2.reshape(1, 512), W3.reshape(1, 512), b3.reshape(1, 1))
